# baseline (speedup 1.0000x reference)
import jax
import jax.numpy as jnp
from jax import lax
from jax.experimental import pallas as pl
from jax.experimental.pallas import tpu as pltpu
from jax.experimental.pallas import tpu_sc as plsc

N_NODES = 10000
N_EDGES = 320000
D = 128
CENTER_NUM = 16
N_CLASSES = 40

N_PAD = 10240
CHUNK = 128
NUM_WORKERS = 32
CHUNKS_PER_TILE = 80
NUM_CHUNKS = CHUNKS_PER_TILE * NUM_WORKERS
E_SC_PAD = NUM_CHUNKS * CHUNK
ROWS_PER_TILE = N_PAD // 16
TRASH_BIN = N_NODES + 16

HROWS = 64
E_HPAD = 2560 * CHUNK


def _sc_aggregate_body(h_hbm, ei_hbm, part_out,
                       acc_sh, sd_v, sd1_v, rows_v, rows1_v, zrow_v,
                       sem, sem1):
    c = lax.axis_index("c")
    s = lax.axis_index("s")
    w = c * 16 + s

    zero16 = jnp.zeros((16,), jnp.float32)

    def fill_const(i, carry):
        for q in range(D // 16):
            zrow_v[i, pl.ds(q * 16, 16)] = zero16
        return carry
    lax.fori_loop(0, 64, fill_const, 0)

    base_row = s * ROWS_PER_TILE
    for q in range(ROWS_PER_TILE // 64):
        pltpu.sync_copy(zrow_v, acc_sh.at[pl.ds(base_row + q * 64, 64)])

    plsc.subcore_barrier()

    def chunk_body(p, carry):
        ca = w + NUM_WORKERS * (4 * p)
        cb = ca + NUM_WORKERS
        cc = cb + NUM_WORKERS
        cd = cc + NUM_WORKERS
        pltpu.sync_copy(ei_hbm.at[ca], sd_v)
        pltpu.async_copy(h_hbm.at[sd_v.at[0]], rows_v, sem).wait()
        pltpu.sync_copy(ei_hbm.at[cb], sd1_v)
        gb = pltpu.async_copy(h_hbm.at[sd1_v.at[0]], rows1_v, sem1)
        pltpu.sync_copy(rows_v, acc_sh.at[sd_v.at[1]], add=True)
        gb.wait()
        pltpu.sync_copy(ei_hbm.at[cc], sd_v)
        gc = pltpu.async_copy(h_hbm.at[sd_v.at[0]], rows_v, sem)
        pltpu.sync_copy(rows1_v, acc_sh.at[sd1_v.at[1]], add=True)
        gc.wait()
        pltpu.sync_copy(ei_hbm.at[cd], sd1_v)
        gd = pltpu.async_copy(h_hbm.at[sd1_v.at[0]], rows1_v, sem1)
        pltpu.sync_copy(rows_v, acc_sh.at[sd_v.at[1]], add=True)
        gd.wait()
        pltpu.sync_copy(rows1_v, acc_sh.at[sd1_v.at[1]], add=True)
        return carry

    lax.fori_loop(0, CHUNKS_PER_TILE // 4, chunk_body, 0)

    plsc.subcore_barrier()

    pltpu.sync_copy(acc_sh.at[pl.ds(base_row, ROWS_PER_TILE)],
                    part_out.at[c, pl.ds(base_row, ROWS_PER_TILE)])


def _sc_aggregate(h, ei_chunks):
    mesh = plsc.VectorSubcoreMesh(core_axis_name="c", subcore_axis_name="s")
    return pl.kernel(
        _sc_aggregate_body,
        out_type=jax.ShapeDtypeStruct((2, N_PAD, D), jnp.float32),
        mesh=mesh,
        scratch_types=[
            pltpu.VMEM_SHARED((N_PAD, D), jnp.float32),
            pltpu.VMEM((2, CHUNK), jnp.int32),
            pltpu.VMEM((2, CHUNK), jnp.int32),
            pltpu.VMEM((CHUNK, D), jnp.float32),
            pltpu.VMEM((CHUNK, D), jnp.float32),
            pltpu.VMEM((64, D), jnp.float32),
            pltpu.SemaphoreType.DMA,
            pltpu.SemaphoreType.DMA,
        ],
    )(h, ei_chunks)


def _tc_hist_body(src_ref, dst_ref, cnt_ref, flag_ref):
    i = pl.program_id(0)
    s = src_ref[...]
    d = dst_ref[...]
    hi = d >> 7
    lo = d & 127
    oh_hi = (lax.broadcasted_iota(jnp.int32, (HROWS, 128, N_PAD // 128), 2)
             == hi[:, :, None]).astype(jnp.float32)
    oh_lo = (lax.broadcasted_iota(jnp.int32, (HROWS, 128, 128), 2)
             == lo[:, :, None]).astype(jnp.float32)
    per_u = lax.dot_general(oh_hi, oh_lo, (((1,), (1,)), ((0,), (0,))),
                            preferred_element_type=jnp.float32)
    contrib = jnp.sum(per_u, axis=0)
    fcontrib = jnp.sum((s == d).astype(jnp.float32))

    @pl.when(i == 0)
    def _():
        cnt_ref[...] = contrib
        flag_ref[...] = jnp.full((8, 128), fcontrib, jnp.float32)

    @pl.when(i > 0)
    def _():
        cnt_ref[...] = cnt_ref[...] + contrib
        flag_ref[...] = flag_ref[...] + fcontrib


def _tc_hist(src2d, dst2d):
    grid = (E_HPAD // 128 // HROWS,)
    return pl.pallas_call(
        _tc_hist_body,
        grid=grid,
        in_specs=[
            pl.BlockSpec((HROWS, 128), lambda i: (i, 0)),
            pl.BlockSpec((HROWS, 128), lambda i: (i, 0)),
        ],
        out_specs=[
            pl.BlockSpec((N_PAD // 128, 128), lambda i: (0, 0)),
            pl.BlockSpec((8, 128), lambda i: (0, 0)),
        ],
        out_shape=[
            jax.ShapeDtypeStruct((N_PAD // 128, 128), jnp.float32),
            jax.ShapeDtypeStruct((8, 128), jnp.float32),
        ],
    )(src2d, dst2d)


def _tc_dense_body(part_ref, cnt_ref, flag_ref, h_ref, ws_ref, wt_ref, out_ref):
    psum = part_ref[0] + part_ref[1]
    cnt = cnt_ref[...]
    loop_total = jnp.sum(flag_ref[...])
    loop_w = jnp.where(loop_total > 0.0, 0.0, 1.0)

    hm = (psum + loop_w * h_ref[...]) / jnp.maximum(cnt + loop_w, 1.0)

    logits = lax.dot_general(hm, ws_ref[...], (((1,), (1,)), ((), ())),
                             preferred_element_type=jnp.float32)
    maxv = jnp.max(logits, axis=1, keepdims=True)
    iota = lax.broadcasted_iota(jnp.int32, logits.shape, 1)
    idx = jnp.min(jnp.where(logits == maxv, iota, CENTER_NUM),
                  axis=1, keepdims=True)

    allout = lax.dot_general(hm, wt_ref[...], (((1,), (1,)), ((), ())),
                             preferred_element_type=jnp.float32)
    acc = jnp.zeros((out_ref.shape[0], N_CLASSES), jnp.float32)
    for k in range(CENTER_NUM):
        acc = acc + jnp.where(idx == k,
                              allout[:, k * N_CLASSES:(k + 1) * N_CLASSES],
                              0.0)
    out_ref[...] = acc


def _tc_dense(partial, cnt_flat, flag, h_pad, W_structure, Wt_flat):
    B = 512
    grid = (N_PAD // B,)
    return pl.pallas_call(
        _tc_dense_body,
        grid=grid,
        in_specs=[
            pl.BlockSpec((2, B, D), lambda i: (0, i, 0)),
            pl.BlockSpec((B, 1), lambda i: (i, 0)),
            pl.BlockSpec((8, 128), lambda i: (0, 0)),
            pl.BlockSpec((B, D), lambda i: (i, 0)),
            pl.BlockSpec((CENTER_NUM, D), lambda i: (0, 0)),
            pl.BlockSpec((CENTER_NUM * N_CLASSES, D), lambda i: (0, 0)),
        ],
        out_specs=pl.BlockSpec((B, N_CLASSES), lambda i: (i, 0)),
        out_shape=jax.ShapeDtypeStruct((N_PAD, N_CLASSES), jnp.float32),
    )(partial, cnt_flat, flag, h_pad, W_structure, Wt_flat)


def kernel(h, edge_index, W_structure, W_task):
    n_sc_extra = E_SC_PAD - N_EDGES
    sc_pad = jnp.stack([
        jnp.zeros((n_sc_extra,), edge_index.dtype),
        jnp.full((n_sc_extra,), TRASH_BIN, edge_index.dtype),
    ])
    ei_chunks = jnp.transpose(
        jnp.concatenate([edge_index, sc_pad], axis=1)
        .reshape(2, NUM_CHUNKS, CHUNK), (1, 0, 2))
    partial = _sc_aggregate(h, ei_chunks)

    n_extra = E_HPAD - N_EDGES
    srcp = jnp.concatenate(
        [edge_index[0], jnp.zeros((n_extra,), edge_index.dtype)]
    ).reshape(E_HPAD // 128, 128)
    dstp = jnp.concatenate(
        [edge_index[1], jnp.full((n_extra,), TRASH_BIN, edge_index.dtype)]
    ).reshape(E_HPAD // 128, 128)
    cnt, flag = _tc_hist(srcp, dstp)
    cnt_flat = cnt.reshape(N_PAD, 1)

    h_pad = jnp.pad(h, ((0, N_PAD - N_NODES), (0, 0)))
    Wt_flat = W_task.reshape(CENTER_NUM * N_CLASSES, D)
    out = _tc_dense(partial, cnt_flat, flag, h_pad, W_structure, Wt_flat)
    return out[:N_NODES]
